# baseline (speedup 1.0000x reference)
import functools

import jax
from jax import lax
import jax.numpy as jnp
from jax.experimental import pallas as pl
from jax.experimental.pallas import tpu as pltpu
from jax.experimental.pallas import tpu_sc as plsc

N = 10000
DIM = 32
STEPS = 6
EB = 1024
NB = 2000

NC = 2
NS = 16
NW = NC * NS
CH = 128
NCH = 40
EPW = CH * NCH
EPAD = NW * EPW
NT = 10240


def _prep_nodes_body(x_ref, w_ref, b_ref, o_ref):
    o_ref[...] = jax.nn.relu(
        jnp.dot(x_ref[...], w_ref[...], preferred_element_type=jnp.float32)
        + b_ref[...])


def _prep_edges_body(ea_ref, w1_ref, b1_ref, o_ref):
    o_ref[...] = jax.nn.relu(
        jnp.dot(ea_ref[...], w1_ref[...], preferred_element_type=jnp.float32)
        + b1_ref[...])


def _msg_body(xj_ref, h_ref, w2t_ref, x1_ref, b2_ref, o_ref):
    xj = xj_ref[...]
    rep = lax.broadcasted_iota(jnp.int32, (EB, DIM * DIM), 1) // DIM
    xr = jnp.take_along_axis(xj, rep, axis=1)
    w = jnp.dot(h_ref[...], w2t_ref[...], preferred_element_type=jnp.float32)
    prod = (w + b2_ref[...]) * xr
    acc = prod[:, 0:128]
    for g in range(1, 8):
        acc = acc + prod[:, g * 128:(g + 1) * 128]
    o_ref[...] = (acc[:, 0:DIM] + acc[:, DIM:2 * DIM]
                  + acc[:, 2 * DIM:3 * DIM] + acc[:, 3 * DIM:4 * DIM])


def _update_body(p0_ref, p1_ref, deg_ref, out_ref, root_ref, cb_ref,
                 wih_ref, whh_ref, bih_ref, bhh_ref, new_ref):
    deg = jnp.maximum(deg_ref[...], 1.0)
    agg = (p0_ref[0] + p1_ref[0]) / deg
    out = out_ref[...]
    outb = out.astype(jnp.bfloat16)
    m = jax.nn.relu(
        agg + jnp.dot(outb, root_ref[...], preferred_element_type=jnp.float32)
        + cb_ref[...])
    gi = jnp.dot(m.astype(jnp.bfloat16), wih_ref[...],
                 preferred_element_type=jnp.float32) + bih_ref[...]
    gh = jnp.dot(outb, whh_ref[...],
                 preferred_element_type=jnp.float32) + bhh_ref[...]
    r = jax.nn.sigmoid(gi[:, 0:DIM] + gh[:, 0:DIM])
    z = jax.nn.sigmoid(gi[:, DIM:2 * DIM] + gh[:, DIM:2 * DIM])
    n = jnp.tanh(gi[:, 2 * DIM:3 * DIM] + r * gh[:, 2 * DIM:3 * DIM])
    new_ref[...] = (1.0 - z) * n + z * out


_SC_MESH = plsc.VectorSubcoreMesh(core_axis_name="c", subcore_axis_name="s")


def _gather_body(table_hbm, idx_hbm, xj_hbm, idx_v, buf0, buf1, sem0, sem1):
    wid = lax.axis_index("s") * NC + lax.axis_index("c")
    base = wid * EPW
    pltpu.sync_copy(idx_hbm.at[wid], idx_v)
    pltpu.async_copy(table_hbm.at[idx_v.at[0]], buf0, sem0)

    def body(i, carry):
        j0 = 2 * i
        h1 = pltpu.async_copy(table_hbm.at[idx_v.at[j0 + 1]], buf1, sem1)
        pltpu.make_async_copy(table_hbm.at[pl.ds(0, CH)], buf0, sem0).wait()
        pltpu.sync_copy(buf0, xj_hbm.at[pl.ds(base + j0 * CH, CH)])

        @pl.when(j0 + 2 < NCH)
        def _():
            pltpu.async_copy(table_hbm.at[idx_v.at[j0 + 2]], buf0, sem0)

        h1.wait()
        pltpu.sync_copy(buf1, xj_hbm.at[pl.ds(base + (j0 + 1) * CH, CH)])
        return carry

    lax.fori_loop(0, NCH // 2, body, 0)


@functools.partial(
    pl.kernel, mesh=_SC_MESH,
    out_type=jax.ShapeDtypeStruct((EPAD, DIM), jnp.float32),
    scratch_types=[
        pltpu.VMEM((NCH, CH), jnp.int32),
        pltpu.VMEM((CH, DIM), jnp.float32),
        pltpu.VMEM((CH, DIM), jnp.float32),
        pltpu.SemaphoreType.DMA,
        pltpu.SemaphoreType.DMA,
    ],
    compiler_params=pltpu.CompilerParams(use_tc_tiling_on_sc=False),
)
def _sc_gather(table_hbm, idx_hbm, xj_hbm, idx_v, buf0, buf1, sem0, sem1):
    _gather_body(table_hbm, idx_hbm, xj_hbm, idx_v, buf0, buf1, sem0, sem1)


_ROWS_PER_TILE = NT // NS


def _scatter_body(msg_hbm, idx_hbm, zero_hbm, part_hbm, idx_v, buf0, buf1,
                  out_v, sem0, sem1, acc):
    cid = lax.axis_index("c")
    sid = lax.axis_index("s")
    wid = sid * NC + cid
    base = wid * EPW
    r0 = sid * _ROWS_PER_TILE
    pltpu.sync_copy(idx_hbm.at[wid], idx_v)
    pltpu.sync_copy(zero_hbm.at[pl.ds(r0, _ROWS_PER_TILE)],
                    acc.at[pl.ds(r0, _ROWS_PER_TILE)])
    plsc.subcore_barrier()
    pltpu.async_copy(msg_hbm.at[pl.ds(base, CH)], buf0, sem0)

    def body(i, carry):
        j0 = 2 * i
        h1 = pltpu.async_copy(
            msg_hbm.at[pl.ds(base + (j0 + 1) * CH, CH)], buf1, sem1)
        pltpu.make_async_copy(msg_hbm.at[pl.ds(0, CH)], buf0, sem0).wait()
        pltpu.sync_copy(buf0, acc.at[idx_v.at[j0]], add=True)

        @pl.when(j0 + 2 < NCH)
        def _():
            pltpu.async_copy(
                msg_hbm.at[pl.ds(base + (j0 + 2) * CH, CH)], buf0, sem0)

        h1.wait()
        pltpu.sync_copy(buf1, acc.at[idx_v.at[j0 + 1]], add=True)
        return carry

    lax.fori_loop(0, NCH // 2, body, 0)
    plsc.subcore_barrier()
    pltpu.sync_copy(acc.at[pl.ds(r0, _ROWS_PER_TILE)], out_v)
    pltpu.sync_copy(out_v, part_hbm.at[cid].at[pl.ds(r0, _ROWS_PER_TILE)])


@functools.partial(
    pl.kernel, mesh=_SC_MESH,
    out_type=jax.ShapeDtypeStruct((NC, NT, DIM), jnp.float32),
    scratch_types=[
        pltpu.VMEM((NCH, CH), jnp.int32),
        pltpu.VMEM((CH, DIM), jnp.float32),
        pltpu.VMEM((CH, DIM), jnp.float32),
        pltpu.VMEM((_ROWS_PER_TILE, DIM), jnp.float32),
        pltpu.SemaphoreType.DMA,
        pltpu.SemaphoreType.DMA,
        pltpu.VMEM_SHARED((NT, DIM), jnp.float32),
    ],
    compiler_params=pltpu.CompilerParams(use_tc_tiling_on_sc=False),
)
def _sc_scatter(msg_hbm, idx_hbm, zero_hbm, part_hbm, idx_v, buf0, buf1,
                out_v, sem0, sem1, acc):
    _scatter_body(msg_hbm, idx_hbm, zero_hbm, part_hbm, idx_v, buf0, buf1,
                  out_v, sem0, sem1, acc)


def _full(shape):
    return pl.BlockSpec(shape, lambda i: (0,) * len(shape))


def _rows(nrows, ncols):
    return pl.BlockSpec((nrows, ncols), lambda i: (i, 0))


def _prep_nodes(xp, fct, fcb):
    return pl.pallas_call(
        _prep_nodes_body,
        grid=(N // NB,),
        in_specs=[_rows(NB, 8), _full((8, DIM)), _full((1, DIM))],
        out_specs=_rows(NB, DIM),
        out_shape=jax.ShapeDtypeStruct((N, DIM), jnp.float32),
    )(xp, fct, fcb)


def _prep_edges(ea_p, w1t, b1, epad):
    return pl.pallas_call(
        _prep_edges_body,
        grid=(epad // EB,),
        in_specs=[_rows(EB, 16), _full((16, DIM)), _full((1, DIM))],
        out_specs=_rows(EB, DIM),
        out_shape=jax.ShapeDtypeStruct((epad, DIM), jnp.float32),
    )(ea_p, w1t, b1)


def _msg(xj, hid, w2t, x1, b2, epad):
    return pl.pallas_call(
        _msg_body,
        grid=(epad // EB,),
        in_specs=[_rows(EB, DIM), _rows(EB, DIM),
                  _full((DIM, DIM * DIM)), _full((DIM, DIM * DIM)),
                  _full((1, DIM * DIM))],
        out_specs=_rows(EB, DIM),
        out_shape=jax.ShapeDtypeStruct((epad, DIM), jnp.float32),
    )(xj, hid, w2t, x1, b2)


def _update(part, deg, out, root, cb, wih, whh, bih, bhh):
    return pl.pallas_call(
        _update_body,
        grid=(N // NB,),
        in_specs=[pl.BlockSpec((1, NB, DIM), lambda i: (0, i, 0)),
                  pl.BlockSpec((1, NB, DIM), lambda i: (1, i, 0)),
                  _rows(NB, 1), _rows(NB, DIM),
                  _full((DIM, DIM)), _full((1, DIM)),
                  _full((DIM, 3 * DIM)), _full((DIM, 3 * DIM)),
                  _full((1, 3 * DIM)), _full((1, 3 * DIM))],
        out_specs=_rows(NB, DIM),
        out_shape=jax.ShapeDtypeStruct((N, DIM), jnp.float32),
    )(part, part, deg, out, root, cb, wih, whh, bih, bhh)


def kernel(x, edge_index, edge_attr, fc_W, fc_b, root, conv_bias,
           en_W1, en_b1, en_W2, en_b2, gru_Wih, gru_Whh, gru_bih, gru_bhh):
    e = edge_index.shape[1]
    src = edge_index[0]
    dst = edge_index[1]
    src_t = jnp.concatenate(
        [src, jnp.zeros((EPAD - e,), src.dtype)]).reshape(NW, NCH, CH)
    dst_t = jnp.concatenate(
        [dst, jnp.full((EPAD - e,), N, dst.dtype)]).reshape(NW, NCH, CH)
    ea_p = jnp.pad(edge_attr, ((0, EPAD - e), (0, 0)))
    xp = jnp.pad(x, ((0, 0), (0, 8 - x.shape[1])))
    fct = jnp.pad(fc_W.T, ((0, 8 - x.shape[1]), (0, 0)))

    out = _prep_nodes(xp, fct, fc_b[None])
    hid = _prep_edges(ea_p, en_W1.T, en_b1[None], EPAD).astype(jnp.bfloat16)
    w2t = en_W2.T.reshape(DIM, DIM * DIM).astype(jnp.bfloat16)
    x1 = jnp.repeat(jnp.eye(DIM, dtype=jnp.bfloat16), DIM, axis=1)
    b2 = en_b2[None]

    zero_nt = jnp.zeros((NT, DIM), jnp.float32)
    ones_e = jnp.ones((EPAD, DIM), jnp.float32)
    dpart = _sc_scatter(ones_e, dst_t, zero_nt)
    deg = (dpart[0, :N, 0] + dpart[1, :N, 0])[:, None]
    rootb = root.astype(jnp.bfloat16)
    wih = gru_Wih.T.astype(jnp.bfloat16)
    whh = gru_Whh.T.astype(jnp.bfloat16)

    for _ in range(STEPS):
        xj = _sc_gather(out, src_t)
        msg = _msg(xj, hid, w2t, x1, b2, EPAD)
        part = _sc_scatter(msg, dst_t, zero_nt)
        out = _update(part, deg, out, rootb, conv_bias[None],
                      wih, whh, gru_bih[None], gru_bhh[None])
    return out


# speedup vs baseline: 1.5969x; 1.5969x over previous
import functools

import jax
from jax import lax
import jax.numpy as jnp
from jax.experimental import pallas as pl
from jax.experimental.pallas import tpu as pltpu
from jax.experimental.pallas import tpu_sc as plsc

N = 10000
DIM = 32
STEPS = 6
EB = 1024
NB = 2000

NC = 2
NS = 16
NW = NC * NS
CH = 128
NCH = 40
EPW = CH * NCH
EPAD = NW * EPW
NT = 10240


def _prep_nodes_body(x_ref, w_ref, b_ref, o_ref):
    o_ref[...] = jax.nn.relu(
        jnp.dot(x_ref[...], w_ref[...], preferred_element_type=jnp.float32)
        + b_ref[...])


def _prep_edges_body(ea_ref, w1_ref, b1_ref, o_ref):
    o_ref[...] = jax.nn.relu(
        jnp.dot(ea_ref[...], w1_ref[...], preferred_element_type=jnp.float32)
        + b1_ref[...])


def _msg_body(xj_ref, h_ref, w2t_ref, x1_ref, b2_ref, o_ref):
    xj = xj_ref[...]
    xr = jnp.dot(xj.astype(jnp.bfloat16), x1_ref[...],
                 preferred_element_type=jnp.float32)
    w = jnp.dot(h_ref[...], w2t_ref[...], preferred_element_type=jnp.float32)
    prod = (w + b2_ref[...]) * xr
    acc = prod[:, 0:128]
    for g in range(1, 8):
        acc = acc + prod[:, g * 128:(g + 1) * 128]
    o_ref[...] = (acc[:, 0:DIM] + acc[:, DIM:2 * DIM]
                  + acc[:, 2 * DIM:3 * DIM] + acc[:, 3 * DIM:4 * DIM])


def _update_body(p0_ref, p1_ref, deg_ref, out_ref, root_ref, cb_ref,
                 wih_ref, whh_ref, bih_ref, bhh_ref, new_ref):
    deg = jnp.maximum(deg_ref[...], 1.0)
    agg = (p0_ref[0] + p1_ref[0]) / deg
    out = out_ref[...]
    outb = out.astype(jnp.bfloat16)
    m = jax.nn.relu(
        agg + jnp.dot(outb, root_ref[...], preferred_element_type=jnp.float32)
        + cb_ref[...])
    gi = jnp.dot(m.astype(jnp.bfloat16), wih_ref[...],
                 preferred_element_type=jnp.float32) + bih_ref[...]
    gh = jnp.dot(outb, whh_ref[...],
                 preferred_element_type=jnp.float32) + bhh_ref[...]
    r = jax.nn.sigmoid(gi[:, 0:DIM] + gh[:, 0:DIM])
    z = jax.nn.sigmoid(gi[:, DIM:2 * DIM] + gh[:, DIM:2 * DIM])
    n = jnp.tanh(gi[:, 2 * DIM:3 * DIM] + r * gh[:, 2 * DIM:3 * DIM])
    new_ref[...] = (1.0 - z) * n + z * out


_SC_MESH = plsc.VectorSubcoreMesh(core_axis_name="c", subcore_axis_name="s")


def _gather_body(table_hbm, idx_hbm, xj_hbm, idx_v, buf0, buf1, sem0, sem1):
    wid = lax.axis_index("s") * NC + lax.axis_index("c")
    base = wid * EPW
    pltpu.sync_copy(idx_hbm.at[wid], idx_v)
    pltpu.async_copy(table_hbm.at[idx_v.at[0]], buf0, sem0)

    def body(i, carry):
        j0 = 2 * i
        h1 = pltpu.async_copy(table_hbm.at[idx_v.at[j0 + 1]], buf1, sem1)
        pltpu.make_async_copy(table_hbm.at[pl.ds(0, CH)], buf0, sem0).wait()
        pltpu.sync_copy(buf0, xj_hbm.at[pl.ds(base + j0 * CH, CH)])

        @pl.when(j0 + 2 < NCH)
        def _():
            pltpu.async_copy(table_hbm.at[idx_v.at[j0 + 2]], buf0, sem0)

        h1.wait()
        pltpu.sync_copy(buf1, xj_hbm.at[pl.ds(base + (j0 + 1) * CH, CH)])
        return carry

    lax.fori_loop(0, NCH // 2, body, 0)


@functools.partial(
    pl.kernel, mesh=_SC_MESH,
    out_type=jax.ShapeDtypeStruct((EPAD, DIM), jnp.float32),
    scratch_types=[
        pltpu.VMEM((NCH, CH), jnp.int32),
        pltpu.VMEM((CH, DIM), jnp.float32),
        pltpu.VMEM((CH, DIM), jnp.float32),
        pltpu.SemaphoreType.DMA,
        pltpu.SemaphoreType.DMA,
    ],
    compiler_params=pltpu.CompilerParams(use_tc_tiling_on_sc=False),
)
def _sc_gather(table_hbm, idx_hbm, xj_hbm, idx_v, buf0, buf1, sem0, sem1):
    _gather_body(table_hbm, idx_hbm, xj_hbm, idx_v, buf0, buf1, sem0, sem1)


_ROWS_PER_TILE = NT // NS


def _scatter_body(msg_hbm, idx_hbm, zero_hbm, part_hbm, idx_v, buf0, buf1,
                  out_v, sem0, sem1, acc):
    cid = lax.axis_index("c")
    sid = lax.axis_index("s")
    wid = sid * NC + cid
    base = wid * EPW
    r0 = sid * _ROWS_PER_TILE
    pltpu.sync_copy(idx_hbm.at[wid], idx_v)
    pltpu.sync_copy(zero_hbm.at[pl.ds(r0, _ROWS_PER_TILE)],
                    acc.at[pl.ds(r0, _ROWS_PER_TILE)])
    plsc.subcore_barrier()
    pltpu.async_copy(msg_hbm.at[pl.ds(base, CH)], buf0, sem0)

    def body(i, carry):
        j0 = 2 * i
        h1 = pltpu.async_copy(
            msg_hbm.at[pl.ds(base + (j0 + 1) * CH, CH)], buf1, sem1)
        pltpu.make_async_copy(msg_hbm.at[pl.ds(0, CH)], buf0, sem0).wait()
        pltpu.sync_copy(buf0, acc.at[idx_v.at[j0]], add=True)

        @pl.when(j0 + 2 < NCH)
        def _():
            pltpu.async_copy(
                msg_hbm.at[pl.ds(base + (j0 + 2) * CH, CH)], buf0, sem0)

        h1.wait()
        pltpu.sync_copy(buf1, acc.at[idx_v.at[j0 + 1]], add=True)
        return carry

    lax.fori_loop(0, NCH // 2, body, 0)
    plsc.subcore_barrier()
    pltpu.sync_copy(acc.at[pl.ds(r0, _ROWS_PER_TILE)], out_v)
    pltpu.sync_copy(out_v, part_hbm.at[cid].at[pl.ds(r0, _ROWS_PER_TILE)])


@functools.partial(
    pl.kernel, mesh=_SC_MESH,
    out_type=jax.ShapeDtypeStruct((NC, NT, DIM), jnp.float32),
    scratch_types=[
        pltpu.VMEM((NCH, CH), jnp.int32),
        pltpu.VMEM((CH, DIM), jnp.float32),
        pltpu.VMEM((CH, DIM), jnp.float32),
        pltpu.VMEM((_ROWS_PER_TILE, DIM), jnp.float32),
        pltpu.SemaphoreType.DMA,
        pltpu.SemaphoreType.DMA,
        pltpu.VMEM_SHARED((NT, DIM), jnp.float32),
    ],
    compiler_params=pltpu.CompilerParams(use_tc_tiling_on_sc=False),
)
def _sc_scatter(msg_hbm, idx_hbm, zero_hbm, part_hbm, idx_v, buf0, buf1,
                out_v, sem0, sem1, acc):
    _scatter_body(msg_hbm, idx_hbm, zero_hbm, part_hbm, idx_v, buf0, buf1,
                  out_v, sem0, sem1, acc)


def _full(shape):
    return pl.BlockSpec(shape, lambda i: (0,) * len(shape))


def _rows(nrows, ncols):
    return pl.BlockSpec((nrows, ncols), lambda i: (i, 0))


def _prep_nodes(xp, fct, fcb):
    return pl.pallas_call(
        _prep_nodes_body,
        grid=(N // NB,),
        in_specs=[_rows(NB, 8), _full((8, DIM)), _full((1, DIM))],
        out_specs=_rows(NB, DIM),
        out_shape=jax.ShapeDtypeStruct((N, DIM), jnp.float32),
    )(xp, fct, fcb)


def _prep_edges(ea_p, w1t, b1, epad):
    return pl.pallas_call(
        _prep_edges_body,
        grid=(epad // EB,),
        in_specs=[_rows(EB, 16), _full((16, DIM)), _full((1, DIM))],
        out_specs=_rows(EB, DIM),
        out_shape=jax.ShapeDtypeStruct((epad, DIM), jnp.float32),
    )(ea_p, w1t, b1)


def _msg(xj, hid, w2t, x1, b2, epad):
    return pl.pallas_call(
        _msg_body,
        grid=(epad // EB,),
        in_specs=[_rows(EB, DIM), _rows(EB, DIM),
                  _full((DIM, DIM * DIM)), _full((DIM, DIM * DIM)),
                  _full((1, DIM * DIM))],
        out_specs=_rows(EB, DIM),
        out_shape=jax.ShapeDtypeStruct((epad, DIM), jnp.float32),
    )(xj, hid, w2t, x1, b2)


def _update(part, deg, out, root, cb, wih, whh, bih, bhh):
    return pl.pallas_call(
        _update_body,
        grid=(N // NB,),
        in_specs=[pl.BlockSpec((1, NB, DIM), lambda i: (0, i, 0)),
                  pl.BlockSpec((1, NB, DIM), lambda i: (1, i, 0)),
                  _rows(NB, 1), _rows(NB, DIM),
                  _full((DIM, DIM)), _full((1, DIM)),
                  _full((DIM, 3 * DIM)), _full((DIM, 3 * DIM)),
                  _full((1, 3 * DIM)), _full((1, 3 * DIM))],
        out_specs=_rows(NB, DIM),
        out_shape=jax.ShapeDtypeStruct((N, DIM), jnp.float32),
    )(part, part, deg, out, root, cb, wih, whh, bih, bhh)


def kernel(x, edge_index, edge_attr, fc_W, fc_b, root, conv_bias,
           en_W1, en_b1, en_W2, en_b2, gru_Wih, gru_Whh, gru_bih, gru_bhh):
    e = edge_index.shape[1]
    src = edge_index[0]
    dst = edge_index[1]
    src_t = jnp.concatenate(
        [src, jnp.zeros((EPAD - e,), src.dtype)]).reshape(NW, NCH, CH)
    dst_t = jnp.concatenate(
        [dst, jnp.full((EPAD - e,), N, dst.dtype)]).reshape(NW, NCH, CH)
    ea_p = jnp.pad(edge_attr, ((0, EPAD - e), (0, 0)))
    xp = jnp.pad(x, ((0, 0), (0, 8 - x.shape[1])))
    fct = jnp.pad(fc_W.T, ((0, 8 - x.shape[1]), (0, 0)))

    out = _prep_nodes(xp, fct, fc_b[None])
    hid = _prep_edges(ea_p, en_W1.T, en_b1[None], EPAD).astype(jnp.bfloat16)
    w2t = en_W2.T.reshape(DIM, DIM * DIM).astype(jnp.bfloat16)
    x1 = jnp.repeat(jnp.eye(DIM, dtype=jnp.bfloat16), DIM, axis=1)
    b2 = en_b2[None]

    zero_nt = jnp.zeros((NT, DIM), jnp.float32)
    ones_e = jnp.ones((EPAD, DIM), jnp.float32)
    dpart = _sc_scatter(ones_e, dst_t, zero_nt)
    deg = (dpart[0, :N, 0] + dpart[1, :N, 0])[:, None]
    rootb = root.astype(jnp.bfloat16)
    wih = gru_Wih.T.astype(jnp.bfloat16)
    whh = gru_Whh.T.astype(jnp.bfloat16)

    for _ in range(STEPS):
        xj = _sc_gather(out, src_t)
        msg = _msg(xj, hid, w2t, x1, b2, EPAD)
        part = _sc_scatter(msg, dst_t, zero_nt)
        out = _update(part, deg, out, rootb, conv_bias[None],
                      wih, whh, gru_bih[None], gru_bhh[None])
    return out


# speedup vs baseline: 1.6782x; 1.0509x over previous
import functools

import jax
from jax import lax
import jax.numpy as jnp
from jax.experimental import pallas as pl
from jax.experimental.pallas import tpu as pltpu
from jax.experimental.pallas import tpu_sc as plsc

N = 10000
DIM = 32
STEPS = 6
EB = 2048
NB = 2000

NC = 2
NS = 16
NW = NC * NS
CH = 128
NCH = 40
EPW = CH * NCH
EPAD = NW * EPW
NT = 10240


def _prep_nodes_body(x_ref, w_ref, b_ref, o_ref):
    o_ref[...] = jax.nn.relu(
        jnp.dot(x_ref[...], w_ref[...], preferred_element_type=jnp.float32)
        + b_ref[...])


def _prep_edges_body(ea_ref, w1_ref, b1_ref, o_ref):
    o_ref[...] = jax.nn.relu(
        jnp.dot(ea_ref[...], w1_ref[...], preferred_element_type=jnp.float32)
        + b1_ref[...])


def _msg_body(xj_ref, h_ref, w2t_ref, x1_ref, b2_ref, o_ref):
    xj = xj_ref[...]
    xr = jnp.dot(xj.astype(jnp.bfloat16), x1_ref[...],
                 preferred_element_type=jnp.float32)
    w = jnp.dot(h_ref[...], w2t_ref[...], preferred_element_type=jnp.float32)
    prod = (w + b2_ref[...]) * xr
    acc = prod[:, 0:128]
    for g in range(1, 8):
        acc = acc + prod[:, g * 128:(g + 1) * 128]
    o_ref[...] = (acc[:, 0:DIM] + acc[:, DIM:2 * DIM]
                  + acc[:, 2 * DIM:3 * DIM] + acc[:, 3 * DIM:4 * DIM])


def _update_body(p0_ref, p1_ref, deg_ref, out_ref, root_ref, cb_ref,
                 wih_ref, whh_ref, bih_ref, bhh_ref, new_ref):
    deg = jnp.maximum(deg_ref[...], 1.0)
    agg = (p0_ref[0] + p1_ref[0]) / deg
    out = out_ref[...]
    outb = out.astype(jnp.bfloat16)
    m = jax.nn.relu(
        agg + jnp.dot(outb, root_ref[...], preferred_element_type=jnp.float32)
        + cb_ref[...])
    gi = jnp.dot(m.astype(jnp.bfloat16), wih_ref[...],
                 preferred_element_type=jnp.float32) + bih_ref[...]
    gh = jnp.dot(outb, whh_ref[...],
                 preferred_element_type=jnp.float32) + bhh_ref[...]
    r = jax.nn.sigmoid(gi[:, 0:DIM] + gh[:, 0:DIM])
    z = jax.nn.sigmoid(gi[:, DIM:2 * DIM] + gh[:, DIM:2 * DIM])
    n = jnp.tanh(gi[:, 2 * DIM:3 * DIM] + r * gh[:, 2 * DIM:3 * DIM])
    new_ref[...] = (1.0 - z) * n + z * out


_SC_MESH = plsc.VectorSubcoreMesh(core_axis_name="c", subcore_axis_name="s")


def _gather_body(table_hbm, idx_hbm, xj_hbm, idx_v, buf0, buf1, sem0, sem1):
    wid = lax.axis_index("s") * NC + lax.axis_index("c")
    base = wid * EPW
    pltpu.sync_copy(idx_hbm.at[wid], idx_v)
    pltpu.async_copy(table_hbm.at[idx_v.at[0]], buf0, sem0)

    def body(i, carry):
        j0 = 2 * i
        h1 = pltpu.async_copy(table_hbm.at[idx_v.at[j0 + 1]], buf1, sem1)
        pltpu.make_async_copy(table_hbm.at[pl.ds(0, CH)], buf0, sem0).wait()
        pltpu.sync_copy(buf0, xj_hbm.at[pl.ds(base + j0 * CH, CH)])

        @pl.when(j0 + 2 < NCH)
        def _():
            pltpu.async_copy(table_hbm.at[idx_v.at[j0 + 2]], buf0, sem0)

        h1.wait()
        pltpu.sync_copy(buf1, xj_hbm.at[pl.ds(base + (j0 + 1) * CH, CH)])
        return carry

    lax.fori_loop(0, NCH // 2, body, 0)


@functools.partial(
    pl.kernel, mesh=_SC_MESH,
    out_type=jax.ShapeDtypeStruct((EPAD, DIM), jnp.float32),
    scratch_types=[
        pltpu.VMEM((NCH, CH), jnp.int32),
        pltpu.VMEM((CH, DIM), jnp.float32),
        pltpu.VMEM((CH, DIM), jnp.float32),
        pltpu.SemaphoreType.DMA,
        pltpu.SemaphoreType.DMA,
    ],
    compiler_params=pltpu.CompilerParams(use_tc_tiling_on_sc=False),
)
def _sc_gather(table_hbm, idx_hbm, xj_hbm, idx_v, buf0, buf1, sem0, sem1):
    _gather_body(table_hbm, idx_hbm, xj_hbm, idx_v, buf0, buf1, sem0, sem1)


_ROWS_PER_TILE = NT // NS


def _scatter_body(msg_hbm, idx_hbm, zero_hbm, part_hbm, idx_v, buf0, buf1,
                  out_v, sem0, sem1, acc):
    cid = lax.axis_index("c")
    sid = lax.axis_index("s")
    wid = sid * NC + cid
    base = wid * EPW
    r0 = sid * _ROWS_PER_TILE
    pltpu.sync_copy(idx_hbm.at[wid], idx_v)
    pltpu.sync_copy(zero_hbm.at[pl.ds(r0, _ROWS_PER_TILE)],
                    acc.at[pl.ds(r0, _ROWS_PER_TILE)])
    plsc.subcore_barrier()
    pltpu.async_copy(msg_hbm.at[pl.ds(base, CH)], buf0, sem0)

    def body(i, carry):
        j0 = 2 * i
        h1 = pltpu.async_copy(
            msg_hbm.at[pl.ds(base + (j0 + 1) * CH, CH)], buf1, sem1)
        pltpu.make_async_copy(msg_hbm.at[pl.ds(0, CH)], buf0, sem0).wait()
        pltpu.sync_copy(buf0, acc.at[idx_v.at[j0]], add=True)

        @pl.when(j0 + 2 < NCH)
        def _():
            pltpu.async_copy(
                msg_hbm.at[pl.ds(base + (j0 + 2) * CH, CH)], buf0, sem0)

        h1.wait()
        pltpu.sync_copy(buf1, acc.at[idx_v.at[j0 + 1]], add=True)
        return carry

    lax.fori_loop(0, NCH // 2, body, 0)
    plsc.subcore_barrier()
    pltpu.sync_copy(acc.at[pl.ds(r0, _ROWS_PER_TILE)], out_v)
    pltpu.sync_copy(out_v, part_hbm.at[cid].at[pl.ds(r0, _ROWS_PER_TILE)])


@functools.partial(
    pl.kernel, mesh=_SC_MESH,
    out_type=jax.ShapeDtypeStruct((NC, NT, DIM), jnp.float32),
    scratch_types=[
        pltpu.VMEM((NCH, CH), jnp.int32),
        pltpu.VMEM((CH, DIM), jnp.float32),
        pltpu.VMEM((CH, DIM), jnp.float32),
        pltpu.VMEM((_ROWS_PER_TILE, DIM), jnp.float32),
        pltpu.SemaphoreType.DMA,
        pltpu.SemaphoreType.DMA,
        pltpu.VMEM_SHARED((NT, DIM), jnp.float32),
    ],
    compiler_params=pltpu.CompilerParams(use_tc_tiling_on_sc=False),
)
def _sc_scatter(msg_hbm, idx_hbm, zero_hbm, part_hbm, idx_v, buf0, buf1,
                out_v, sem0, sem1, acc):
    _scatter_body(msg_hbm, idx_hbm, zero_hbm, part_hbm, idx_v, buf0, buf1,
                  out_v, sem0, sem1, acc)


def _full(shape):
    return pl.BlockSpec(shape, lambda i: (0,) * len(shape))


def _rows(nrows, ncols):
    return pl.BlockSpec((nrows, ncols), lambda i: (i, 0))


def _prep_nodes(xp, fct, fcb):
    return pl.pallas_call(
        _prep_nodes_body,
        grid=(N // NB,),
        in_specs=[_rows(NB, 8), _full((8, DIM)), _full((1, DIM))],
        out_specs=_rows(NB, DIM),
        out_shape=jax.ShapeDtypeStruct((N, DIM), jnp.float32),
    )(xp, fct, fcb)


def _prep_edges(ea_p, w1t, b1, epad):
    return pl.pallas_call(
        _prep_edges_body,
        grid=(epad // EB,),
        in_specs=[_rows(EB, 16), _full((16, DIM)), _full((1, DIM))],
        out_specs=_rows(EB, DIM),
        out_shape=jax.ShapeDtypeStruct((epad, DIM), jnp.float32),
    )(ea_p, w1t, b1)


def _msg(xj, hid, w2t, x1, b2, epad):
    return pl.pallas_call(
        _msg_body,
        grid=(epad // EB,),
        in_specs=[_rows(EB, DIM), _rows(EB, DIM),
                  _full((DIM, DIM * DIM)), _full((DIM, DIM * DIM)),
                  _full((1, DIM * DIM))],
        out_specs=_rows(EB, DIM),
        out_shape=jax.ShapeDtypeStruct((epad, DIM), jnp.float32),
    )(xj, hid, w2t, x1, b2)


def _update(part, deg, out, root, cb, wih, whh, bih, bhh):
    return pl.pallas_call(
        _update_body,
        grid=(N // NB,),
        in_specs=[pl.BlockSpec((1, NB, DIM), lambda i: (0, i, 0)),
                  pl.BlockSpec((1, NB, DIM), lambda i: (1, i, 0)),
                  _rows(NB, 1), _rows(NB, DIM),
                  _full((DIM, DIM)), _full((1, DIM)),
                  _full((DIM, 3 * DIM)), _full((DIM, 3 * DIM)),
                  _full((1, 3 * DIM)), _full((1, 3 * DIM))],
        out_specs=_rows(NB, DIM),
        out_shape=jax.ShapeDtypeStruct((N, DIM), jnp.float32),
    )(part, part, deg, out, root, cb, wih, whh, bih, bhh)


def kernel(x, edge_index, edge_attr, fc_W, fc_b, root, conv_bias,
           en_W1, en_b1, en_W2, en_b2, gru_Wih, gru_Whh, gru_bih, gru_bhh):
    e = edge_index.shape[1]
    src = edge_index[0]
    dst = edge_index[1]
    src_t = jnp.concatenate(
        [src, jnp.zeros((EPAD - e,), src.dtype)]).reshape(NW, NCH, CH)
    dst_t = jnp.concatenate(
        [dst, jnp.full((EPAD - e,), N, dst.dtype)]).reshape(NW, NCH, CH)
    ea_p = jnp.pad(edge_attr, ((0, EPAD - e), (0, 0)))
    xp = jnp.pad(x, ((0, 0), (0, 8 - x.shape[1])))
    fct = jnp.pad(fc_W.T, ((0, 8 - x.shape[1]), (0, 0)))

    out = _prep_nodes(xp, fct, fc_b[None])
    hid = _prep_edges(ea_p, en_W1.T, en_b1[None], EPAD).astype(jnp.bfloat16)
    w2t = en_W2.T.reshape(DIM, DIM * DIM).astype(jnp.bfloat16)
    x1 = jnp.repeat(jnp.eye(DIM, dtype=jnp.bfloat16), DIM, axis=1)
    b2 = en_b2[None]

    zero_nt = jnp.zeros((NT, DIM), jnp.float32)
    ones_e = jnp.ones((EPAD, DIM), jnp.float32)
    dpart = _sc_scatter(ones_e, dst_t, zero_nt)
    deg = (dpart[0, :N, 0] + dpart[1, :N, 0])[:, None]
    rootb = root.astype(jnp.bfloat16)
    wih = gru_Wih.T.astype(jnp.bfloat16)
    whh = gru_Whh.T.astype(jnp.bfloat16)

    for _ in range(STEPS):
        xj = _sc_gather(out, src_t)
        msg = _msg(xj, hid, w2t, x1, b2, EPAD)
        part = _sc_scatter(msg, dst_t, zero_nt)
        out = _update(part, deg, out, rootb, conv_bias[None],
                      wih, whh, gru_bih[None], gru_bhh[None])
    return out


# speedup vs baseline: 1.7220x; 1.0261x over previous
import functools

import jax
from jax import lax
import jax.numpy as jnp
from jax.experimental import pallas as pl
from jax.experimental.pallas import tpu as pltpu
from jax.experimental.pallas import tpu_sc as plsc

N = 10000
DIM = 32
STEPS = 6
EB = 4096
NB = 2000

NC = 2
NS = 16
NW = NC * NS
CH = 128
NCH = 40
EPW = CH * NCH
EPAD = NW * EPW
NT = 10240


def _prep_nodes_body(x_ref, w_ref, b_ref, o_ref):
    o_ref[...] = jax.nn.relu(
        jnp.dot(x_ref[...], w_ref[...], preferred_element_type=jnp.float32)
        + b_ref[...])


def _prep_edges_body(ea_ref, w1_ref, b1_ref, o_ref):
    o_ref[...] = jax.nn.relu(
        jnp.dot(ea_ref[...], w1_ref[...], preferred_element_type=jnp.float32)
        + b1_ref[...])


def _msg_body(xj_ref, h_ref, w2t_ref, x1_ref, b2_ref, o_ref):
    xj = xj_ref[...]
    xr = jnp.dot(xj.astype(jnp.bfloat16), x1_ref[...],
                 preferred_element_type=jnp.float32)
    w = jnp.dot(h_ref[...], w2t_ref[...], preferred_element_type=jnp.float32)
    prod = (w + b2_ref[...]) * xr
    acc = prod[:, 0:128]
    for g in range(1, 8):
        acc = acc + prod[:, g * 128:(g + 1) * 128]
    o_ref[...] = (acc[:, 0:DIM] + acc[:, DIM:2 * DIM]
                  + acc[:, 2 * DIM:3 * DIM] + acc[:, 3 * DIM:4 * DIM])


def _update_body(p0_ref, p1_ref, deg_ref, out_ref, root_ref, cb_ref,
                 wih_ref, whh_ref, bih_ref, bhh_ref, new_ref):
    deg = jnp.maximum(deg_ref[...], 1.0)
    agg = (p0_ref[0] + p1_ref[0]) / deg
    out = out_ref[...]
    outb = out.astype(jnp.bfloat16)
    m = jax.nn.relu(
        agg + jnp.dot(outb, root_ref[...], preferred_element_type=jnp.float32)
        + cb_ref[...])
    gi = jnp.dot(m.astype(jnp.bfloat16), wih_ref[...],
                 preferred_element_type=jnp.float32) + bih_ref[...]
    gh = jnp.dot(outb, whh_ref[...],
                 preferred_element_type=jnp.float32) + bhh_ref[...]
    r = jax.nn.sigmoid(gi[:, 0:DIM] + gh[:, 0:DIM])
    z = jax.nn.sigmoid(gi[:, DIM:2 * DIM] + gh[:, DIM:2 * DIM])
    n = jnp.tanh(gi[:, 2 * DIM:3 * DIM] + r * gh[:, 2 * DIM:3 * DIM])
    new_ref[...] = (1.0 - z) * n + z * out


_SC_MESH = plsc.VectorSubcoreMesh(core_axis_name="c", subcore_axis_name="s")


def _gather_body(table_hbm, idx_hbm, xj_hbm, idx_v, buf0, buf1, sem0, sem1):
    wid = lax.axis_index("s") * NC + lax.axis_index("c")
    base = wid * EPW
    pltpu.sync_copy(idx_hbm.at[wid], idx_v)
    pltpu.async_copy(table_hbm.at[idx_v.at[0]], buf0, sem0)

    def body(i, carry):
        j0 = 2 * i
        h1 = pltpu.async_copy(table_hbm.at[idx_v.at[j0 + 1]], buf1, sem1)
        pltpu.make_async_copy(table_hbm.at[pl.ds(0, CH)], buf0, sem0).wait()
        pltpu.sync_copy(buf0, xj_hbm.at[pl.ds(base + j0 * CH, CH)])

        @pl.when(j0 + 2 < NCH)
        def _():
            pltpu.async_copy(table_hbm.at[idx_v.at[j0 + 2]], buf0, sem0)

        h1.wait()
        pltpu.sync_copy(buf1, xj_hbm.at[pl.ds(base + (j0 + 1) * CH, CH)])
        return carry

    lax.fori_loop(0, NCH // 2, body, 0)


@functools.partial(
    pl.kernel, mesh=_SC_MESH,
    out_type=jax.ShapeDtypeStruct((EPAD, DIM), jnp.float32),
    scratch_types=[
        pltpu.VMEM((NCH, CH), jnp.int32),
        pltpu.VMEM((CH, DIM), jnp.float32),
        pltpu.VMEM((CH, DIM), jnp.float32),
        pltpu.SemaphoreType.DMA,
        pltpu.SemaphoreType.DMA,
    ],
    compiler_params=pltpu.CompilerParams(use_tc_tiling_on_sc=False),
)
def _sc_gather(table_hbm, idx_hbm, xj_hbm, idx_v, buf0, buf1, sem0, sem1):
    _gather_body(table_hbm, idx_hbm, xj_hbm, idx_v, buf0, buf1, sem0, sem1)


_ROWS_PER_TILE = NT // NS


def _scatter_body(msg_hbm, idx_hbm, zero_hbm, part_hbm, idx_v, buf0, buf1,
                  out_v, sem0, sem1, acc):
    cid = lax.axis_index("c")
    sid = lax.axis_index("s")
    wid = sid * NC + cid
    base = wid * EPW
    r0 = sid * _ROWS_PER_TILE
    pltpu.sync_copy(idx_hbm.at[wid], idx_v)
    pltpu.sync_copy(zero_hbm.at[pl.ds(r0, _ROWS_PER_TILE)],
                    acc.at[pl.ds(r0, _ROWS_PER_TILE)])
    plsc.subcore_barrier()
    pltpu.async_copy(msg_hbm.at[pl.ds(base, CH)], buf0, sem0)

    def body(i, carry):
        j0 = 2 * i
        h1 = pltpu.async_copy(
            msg_hbm.at[pl.ds(base + (j0 + 1) * CH, CH)], buf1, sem1)
        pltpu.make_async_copy(msg_hbm.at[pl.ds(0, CH)], buf0, sem0).wait()
        pltpu.sync_copy(buf0, acc.at[idx_v.at[j0]], add=True)

        @pl.when(j0 + 2 < NCH)
        def _():
            pltpu.async_copy(
                msg_hbm.at[pl.ds(base + (j0 + 2) * CH, CH)], buf0, sem0)

        h1.wait()
        pltpu.sync_copy(buf1, acc.at[idx_v.at[j0 + 1]], add=True)
        return carry

    lax.fori_loop(0, NCH // 2, body, 0)
    plsc.subcore_barrier()
    pltpu.sync_copy(acc.at[pl.ds(r0, _ROWS_PER_TILE)], out_v)
    pltpu.sync_copy(out_v, part_hbm.at[cid].at[pl.ds(r0, _ROWS_PER_TILE)])


@functools.partial(
    pl.kernel, mesh=_SC_MESH,
    out_type=jax.ShapeDtypeStruct((NC, NT, DIM), jnp.float32),
    scratch_types=[
        pltpu.VMEM((NCH, CH), jnp.int32),
        pltpu.VMEM((CH, DIM), jnp.float32),
        pltpu.VMEM((CH, DIM), jnp.float32),
        pltpu.VMEM((_ROWS_PER_TILE, DIM), jnp.float32),
        pltpu.SemaphoreType.DMA,
        pltpu.SemaphoreType.DMA,
        pltpu.VMEM_SHARED((NT, DIM), jnp.float32),
    ],
    compiler_params=pltpu.CompilerParams(use_tc_tiling_on_sc=False),
)
def _sc_scatter(msg_hbm, idx_hbm, zero_hbm, part_hbm, idx_v, buf0, buf1,
                out_v, sem0, sem1, acc):
    _scatter_body(msg_hbm, idx_hbm, zero_hbm, part_hbm, idx_v, buf0, buf1,
                  out_v, sem0, sem1, acc)


def _full(shape):
    return pl.BlockSpec(shape, lambda i: (0,) * len(shape))


def _rows(nrows, ncols):
    return pl.BlockSpec((nrows, ncols), lambda i: (i, 0))


def _prep_nodes(xp, fct, fcb):
    return pl.pallas_call(
        _prep_nodes_body,
        grid=(N // NB,),
        in_specs=[_rows(NB, 8), _full((8, DIM)), _full((1, DIM))],
        out_specs=_rows(NB, DIM),
        out_shape=jax.ShapeDtypeStruct((N, DIM), jnp.float32),
    )(xp, fct, fcb)


def _prep_edges(ea_p, w1t, b1, epad):
    return pl.pallas_call(
        _prep_edges_body,
        grid=(epad // EB,),
        in_specs=[_rows(EB, 16), _full((16, DIM)), _full((1, DIM))],
        out_specs=_rows(EB, DIM),
        out_shape=jax.ShapeDtypeStruct((epad, DIM), jnp.float32),
    )(ea_p, w1t, b1)


def _msg(xj, hid, w2t, x1, b2, epad):
    return pl.pallas_call(
        _msg_body,
        grid=(epad // EB,),
        in_specs=[_rows(EB, DIM), _rows(EB, DIM),
                  _full((DIM, DIM * DIM)), _full((DIM, DIM * DIM)),
                  _full((1, DIM * DIM))],
        out_specs=_rows(EB, DIM),
        out_shape=jax.ShapeDtypeStruct((epad, DIM), jnp.float32),
    )(xj, hid, w2t, x1, b2)


def _update(part, deg, out, root, cb, wih, whh, bih, bhh):
    return pl.pallas_call(
        _update_body,
        grid=(N // NB,),
        in_specs=[pl.BlockSpec((1, NB, DIM), lambda i: (0, i, 0)),
                  pl.BlockSpec((1, NB, DIM), lambda i: (1, i, 0)),
                  _rows(NB, 1), _rows(NB, DIM),
                  _full((DIM, DIM)), _full((1, DIM)),
                  _full((DIM, 3 * DIM)), _full((DIM, 3 * DIM)),
                  _full((1, 3 * DIM)), _full((1, 3 * DIM))],
        out_specs=_rows(NB, DIM),
        out_shape=jax.ShapeDtypeStruct((N, DIM), jnp.float32),
    )(part, part, deg, out, root, cb, wih, whh, bih, bhh)


def kernel(x, edge_index, edge_attr, fc_W, fc_b, root, conv_bias,
           en_W1, en_b1, en_W2, en_b2, gru_Wih, gru_Whh, gru_bih, gru_bhh):
    e = edge_index.shape[1]
    src = edge_index[0]
    dst = edge_index[1]
    src_t = jnp.concatenate(
        [src, jnp.zeros((EPAD - e,), src.dtype)]).reshape(NW, NCH, CH)
    dst_t = jnp.concatenate(
        [dst, jnp.full((EPAD - e,), N, dst.dtype)]).reshape(NW, NCH, CH)
    ea_p = jnp.pad(edge_attr, ((0, EPAD - e), (0, 0)))
    xp = jnp.pad(x, ((0, 0), (0, 8 - x.shape[1])))
    fct = jnp.pad(fc_W.T, ((0, 8 - x.shape[1]), (0, 0)))

    out = _prep_nodes(xp, fct, fc_b[None])
    hid = _prep_edges(ea_p, en_W1.T, en_b1[None], EPAD).astype(jnp.bfloat16)
    w2t = en_W2.T.reshape(DIM, DIM * DIM).astype(jnp.bfloat16)
    x1 = jnp.repeat(jnp.eye(DIM, dtype=jnp.bfloat16), DIM, axis=1)
    b2 = en_b2[None]

    zero_nt = jnp.zeros((NT, DIM), jnp.float32)
    ones_e = jnp.ones((EPAD, DIM), jnp.float32)
    dpart = _sc_scatter(ones_e, dst_t, zero_nt)
    deg = (dpart[0, :N, 0] + dpart[1, :N, 0])[:, None]
    rootb = root.astype(jnp.bfloat16)
    wih = gru_Wih.T.astype(jnp.bfloat16)
    whh = gru_Whh.T.astype(jnp.bfloat16)

    for _ in range(STEPS):
        xj = _sc_gather(out, src_t)
        msg = _msg(xj, hid, w2t, x1, b2, EPAD)
        part = _sc_scatter(msg, dst_t, zero_nt)
        out = _update(part, deg, out, rootb, conv_bias[None],
                      wih, whh, gru_bih[None], gru_bhh[None])
    return out
